# baseline (speedup 1.0000x reference)
import jax
import jax.numpy as jnp
from jax.experimental import pallas as pl
from jax.experimental.pallas import tpu as pltpu


def _dispatch_body(wu_ref, wd_ref, pu_ref, pd_ref, om_ref, Ru_ref, Rd_ref,
                   du_ref, dd_ref, ls_ref, sp_ref, obj_ref):
    om = om_ref[...]
    f32 = jnp.float32

    def side(W_ref, p_ref, R_ref, dem, alloc_ref, slack_ref):
        capx = jnp.concatenate([R_ref[...], dem], axis=1)
        hi = capx.astype(jnp.bfloat16)
        lo = (capx - hi.astype(f32)).astype(jnp.bfloat16)
        W = W_ref[...]
        acc = (jnp.dot(hi, W, preferred_element_type=f32)
               + jnp.dot(lo, W, preferred_element_type=f32))
        alloc = jnp.clip(-acc, 0.0, capx)
        cost = jnp.sum(alloc * p_ref[...], axis=1)
        n = alloc_ref.shape[1]
        alloc_ref[...] = alloc[:, :n]
        slack_ref[...] = alloc[:, n:]
        return cost

    cost_up = side(wu_ref, pu_ref, Ru_ref, jnp.maximum(om, 0.0),
                   du_ref, ls_ref)
    cost_dn = side(wd_ref, pd_ref, Rd_ref, jnp.maximum(-om, 0.0),
                   dd_ref, sp_ref)
    obj_ref[...] = (cost_up + cost_dn)[:, None]


def _merit_w(prices, T):
    Gp1 = prices.shape[0]
    N = Gp1 * T
    pr = jnp.repeat(prices, T)
    ir = jnp.repeat(jnp.arange(Gp1), T)
    tr = jnp.tile(jnp.arange(T), Gp1)
    before = ((pr[:, None] < pr[None, :])
              | ((pr[:, None] == pr[None, :]) & (ir[:, None] < ir[None, :])))
    val = before.astype(jnp.float32) - (ir[:, None] == Gp1 - 1)
    W = jnp.where(tr[:, None] == tr[None, :], val, 0.0)
    return W.astype(jnp.bfloat16)


def kernel(R_up, R_dn, omega_true, b_G, voll, vosp, rt_up_ratio, rt_dn_ratio):
    B, G, T = R_up.shape
    GT = G * T
    N = GT + T
    bB = 128
    p_up = jnp.concatenate([(rt_up_ratio * b_G).astype(jnp.float32),
                            voll[None]])
    p_dn = jnp.concatenate([(rt_dn_ratio * b_G).astype(jnp.float32),
                            vosp[None]])
    W_up = _merit_w(p_up, T)
    W_dn = _merit_w(p_dn, T)
    pf_up = jnp.repeat(p_up, T)[None, :]
    pf_dn = jnp.repeat(p_dn, T)[None, :]

    grid = (B // bB,)
    blk = lambda i: (i, 0)
    full = lambda *shape: pl.BlockSpec(shape, lambda i: (0,) * len(shape))
    out = pl.pallas_call(
        _dispatch_body,
        grid=grid,
        in_specs=[
            full(N, N), full(N, N), full(1, N), full(1, N),
            pl.BlockSpec((bB, T), blk),
            pl.BlockSpec((bB, GT), blk),
            pl.BlockSpec((bB, GT), blk),
        ],
        out_specs=[
            pl.BlockSpec((bB, GT), blk),
            pl.BlockSpec((bB, GT), blk),
            pl.BlockSpec((bB, T), blk),
            pl.BlockSpec((bB, T), blk),
            pl.BlockSpec((bB, 1), blk),
        ],
        out_shape=[
            jax.ShapeDtypeStruct((B, GT), jnp.float32),
            jax.ShapeDtypeStruct((B, GT), jnp.float32),
            jax.ShapeDtypeStruct((B, T), jnp.float32),
            jax.ShapeDtypeStruct((B, T), jnp.float32),
            jax.ShapeDtypeStruct((B, 1), jnp.float32),
        ],
        compiler_params=pltpu.CompilerParams(
            dimension_semantics=("parallel",),
            allow_input_fusion=[True] * 7,
            vmem_limit_bytes=60 * 1024 * 1024,
        ),
        name="reserve_rt_dispatch",
    )(W_up, W_dn, pf_up, pf_dn, omega_true,
      R_up.reshape(B, GT), R_dn.reshape(B, GT))
    du, dd, LS, SP, obj = out
    return (du.reshape(B, G, T), dd.reshape(B, G, T), LS, SP, obj.reshape(B))


# speedup vs baseline: 1.0171x; 1.0171x over previous
import jax
import jax.numpy as jnp
from jax.experimental import pallas as pl
from jax.experimental.pallas import tpu as pltpu


def _dispatch_body(wu_ref, wd_ref, pu_ref, pd_ref, om_ref, Ru_ref, Rd_ref,
                   du_ref, dd_ref, ls_ref, sp_ref, obj_ref):
    om = om_ref[...]
    f32 = jnp.float32

    def side(W_ref, p_ref, R_ref, dem, alloc_ref, slack_ref):
        capx = jnp.concatenate([R_ref[...], dem], axis=1)
        hi = capx.astype(jnp.bfloat16)
        lo = (capx - hi.astype(f32)).astype(jnp.bfloat16)
        W = W_ref[...]
        hl = jnp.concatenate([hi, lo], axis=0)
        acc2 = jnp.dot(hl, W, preferred_element_type=f32)
        b = hi.shape[0]
        acc = acc2[:b] + acc2[b:]
        alloc = jnp.clip(-acc, 0.0, capx)
        cost = jnp.sum(alloc * p_ref[...], axis=1)
        n = alloc_ref.shape[1]
        alloc_ref[...] = alloc[:, :n]
        slack_ref[...] = alloc[:, n:]
        return cost

    cost_up = side(wu_ref, pu_ref, Ru_ref, jnp.maximum(om, 0.0),
                   du_ref, ls_ref)
    cost_dn = side(wd_ref, pd_ref, Rd_ref, jnp.maximum(-om, 0.0),
                   dd_ref, sp_ref)
    obj_ref[...] = (cost_up + cost_dn)[:, None]


def _merit_w(prices, T):
    Gp1 = prices.shape[0]
    N = Gp1 * T
    pr = jnp.repeat(prices, T)
    ir = jnp.repeat(jnp.arange(Gp1), T)
    tr = jnp.tile(jnp.arange(T), Gp1)
    before = ((pr[:, None] < pr[None, :])
              | ((pr[:, None] == pr[None, :]) & (ir[:, None] < ir[None, :])))
    val = before.astype(jnp.float32) - (ir[:, None] == Gp1 - 1)
    W = jnp.where(tr[:, None] == tr[None, :], val, 0.0)
    return W.astype(jnp.bfloat16)


def kernel(R_up, R_dn, omega_true, b_G, voll, vosp, rt_up_ratio, rt_dn_ratio):
    B, G, T = R_up.shape
    GT = G * T
    N = GT + T
    bB = 256
    p_up = jnp.concatenate([(rt_up_ratio * b_G).astype(jnp.float32),
                            voll[None]])
    p_dn = jnp.concatenate([(rt_dn_ratio * b_G).astype(jnp.float32),
                            vosp[None]])
    W_up = _merit_w(p_up, T)
    W_dn = _merit_w(p_dn, T)
    pf_up = jnp.repeat(p_up, T)[None, :]
    pf_dn = jnp.repeat(p_dn, T)[None, :]

    grid = (B // bB,)
    blk = lambda i: (i, 0)
    full = lambda *shape: pl.BlockSpec(shape, lambda i: (0,) * len(shape))
    out = pl.pallas_call(
        _dispatch_body,
        grid=grid,
        in_specs=[
            full(N, N), full(N, N), full(1, N), full(1, N),
            pl.BlockSpec((bB, T), blk),
            pl.BlockSpec((bB, GT), blk),
            pl.BlockSpec((bB, GT), blk),
        ],
        out_specs=[
            pl.BlockSpec((bB, GT), blk),
            pl.BlockSpec((bB, GT), blk),
            pl.BlockSpec((bB, T), blk),
            pl.BlockSpec((bB, T), blk),
            pl.BlockSpec((bB, 1), blk),
        ],
        out_shape=[
            jax.ShapeDtypeStruct((B, GT), jnp.float32),
            jax.ShapeDtypeStruct((B, GT), jnp.float32),
            jax.ShapeDtypeStruct((B, T), jnp.float32),
            jax.ShapeDtypeStruct((B, T), jnp.float32),
            jax.ShapeDtypeStruct((B, 1), jnp.float32),
        ],
        compiler_params=pltpu.CompilerParams(
            dimension_semantics=("parallel",),
            allow_input_fusion=[True] * 7,
            vmem_limit_bytes=60 * 1024 * 1024,
        ),
        name="reserve_rt_dispatch",
    )(W_up, W_dn, pf_up, pf_dn, omega_true,
      R_up.reshape(B, GT), R_dn.reshape(B, GT))
    du, dd, LS, SP, obj = out
    return (du.reshape(B, G, T), dd.reshape(B, G, T), LS, SP, obj.reshape(B))
